# Initial kernel scaffold; baseline (speedup 1.0000x reference)
import jax
import jax.numpy as jnp
from jax.experimental import pallas as pl


def kernel(x, indices):
    raise NotImplementedError("write your pallas kernel here")


# speedup vs baseline: 2.0893x; 2.0893x over previous
import functools

import jax
import jax.numpy as jnp
from jax import lax
from jax.experimental import pallas as pl
from jax.experimental.pallas import tpu as pltpu
from jax.experimental.pallas import tpu_sc as plsc

NB = 16
C = 768
D = 1024
NC = 2
NS = 16
NW = NC * NS
ROWS_PER_W = NB * C // NW
CHUNK = 48
NCHUNK = ROWS_PER_W // CHUNK


def _shuffle_body(x_hbm, idx_hbm, out_hbm, idx_v, buf0, buf1,
                  gsem0, gsem1, ssem0, ssem1):
    wid = lax.axis_index("s") * NC + lax.axis_index("c")
    base = pl.multiple_of(wid * ROWS_PER_W, ROWS_PER_W)
    c0 = pl.multiple_of((wid % 2) * ROWS_PER_W, ROWS_PER_W)
    row_off = (wid // 2) * C

    pltpu.sync_copy(idx_hbm.at[pl.ds(c0, ROWS_PER_W)], idx_v)
    off_vec = jnp.full((16,), row_off, dtype=jnp.int32)
    for j in range(ROWS_PER_W // 16):
        sl = pl.ds(j * 16, 16)
        idx_v[sl] = idx_v[sl] + off_vec

    bufs = (buf0, buf1)
    gsems = (gsem0, gsem1)
    ssems = (ssem0, ssem1)

    def gather(k):
        slot = k % 2
        return pltpu.async_copy(
            x_hbm.at[idx_v.at[pl.ds(k * CHUNK, CHUNK)]], bufs[slot],
            gsems[slot])

    def scatter(k):
        slot = k % 2
        return pltpu.async_copy(
            bufs[slot], out_hbm.at[pl.ds(base + k * CHUNK, CHUNK)],
            ssems[slot])

    gathers = [None, None]
    scatters = [None, None]
    gathers[0] = gather(0)
    for k in range(NCHUNK):
        slot = k % 2
        nslot = (k + 1) % 2
        gathers[slot].wait()
        if k + 1 < NCHUNK:
            if scatters[nslot] is not None:
                scatters[nslot].wait()
            gathers[nslot] = gather(k + 1)
        scatters[slot] = scatter(k)
    scatters[(NCHUNK - 2) % 2].wait()
    scatters[(NCHUNK - 1) % 2].wait()


@jax.jit
def _shuffle(x_flat, indices):
    mesh = plsc.VectorSubcoreMesh(core_axis_name="c", subcore_axis_name="s")
    return pl.kernel(
        _shuffle_body,
        out_type=jax.ShapeDtypeStruct((NB * C, D), jnp.float32),
        mesh=mesh,
        scratch_types=[
            pltpu.VMEM((ROWS_PER_W,), jnp.int32),
            pltpu.VMEM((CHUNK, D), jnp.float32),
            pltpu.VMEM((CHUNK, D), jnp.float32),
            pltpu.SemaphoreType.DMA,
            pltpu.SemaphoreType.DMA,
            pltpu.SemaphoreType.DMA,
            pltpu.SemaphoreType.DMA,
        ],
    )(x_flat, indices)


def kernel(x, indices):
    y = _shuffle(x.reshape(NB * C, D), indices)
    return (y.reshape(NB, C, 32, 32), jnp.zeros((), dtype=x.dtype))


# speedup vs baseline: 2.7045x; 1.2944x over previous
import jax
import jax.numpy as jnp
from jax import lax
from jax.experimental import pallas as pl
from jax.experimental.pallas import tpu as pltpu
from jax.experimental.pallas import tpu_sc as plsc

NB = 16
C = 768
HW = 32 * 32
N = NB * HW
NC = 2
NS = 16
NW = NC * NS
ROWS_PER_W = N // NW
PCHUNK = 32
NCHUNK = ROWS_PER_W // PCHUNK
CG = C // 16


def _shuffle_body(x_hbm, idx_hbm, out_hbm, idx_v, in0, in1, out0, out1,
                  gsem0, gsem1, ssem0, ssem1):
    wid = lax.axis_index("s") * NC + lax.axis_index("c")
    base = wid * ROWS_PER_W

    pltpu.sync_copy(idx_hbm, idx_v)

    ins = (in0, in1)
    outs = (out0, out1)
    gsems = (gsem0, gsem1)
    ssems = (ssem0, ssem1)

    def gather(k):
        s = k % 2
        return pltpu.async_copy(
            x_hbm.at[pl.ds((base + k * PCHUNK) * C, PCHUNK * C)], ins[s],
            gsems[s])

    def scatter(k):
        s = k % 2
        return pltpu.async_copy(
            outs[s], out_hbm.at[pl.ds((base + k * PCHUNK) * C, PCHUNK * C)],
            ssems[s])

    def permute_chunk(in_buf, out_buf):
        def body(j, _):
            idxv = idx_v[pl.ds(j * 16, 16)]
            for p in range(PCHUNK):
                flat = idxv + jnp.full((16,), p * C, dtype=jnp.int32)
                out_buf[pl.ds(j * 16 + p * C, 16)] = (
                    plsc.load_gather(in_buf, [flat]))
            return 0
        lax.fori_loop(0, CG, body, 0)

    gathers = [None, None]
    scatters = [None, None]
    gathers[0] = gather(0)
    for k in range(NCHUNK):
        s = k % 2
        gathers[s].wait()
        if k + 1 < NCHUNK:
            gathers[1 - s] = gather(k + 1)
        if scatters[s] is not None:
            scatters[s].wait()
        permute_chunk(ins[s], outs[s])
        scatters[s] = scatter(k)
    scatters[0].wait()
    scatters[1].wait()


@jax.jit
def _shuffle(xt, indices):
    mesh = plsc.VectorSubcoreMesh(core_axis_name="c", subcore_axis_name="s")
    return pl.kernel(
        _shuffle_body,
        out_type=jax.ShapeDtypeStruct((N * C,), jnp.float32),
        mesh=mesh,
        scratch_types=[
            pltpu.VMEM((C,), jnp.int32),
            pltpu.VMEM((PCHUNK * C,), jnp.float32),
            pltpu.VMEM((PCHUNK * C,), jnp.float32),
            pltpu.VMEM((PCHUNK * C,), jnp.float32),
            pltpu.VMEM((PCHUNK * C,), jnp.float32),
            pltpu.SemaphoreType.DMA,
            pltpu.SemaphoreType.DMA,
            pltpu.SemaphoreType.DMA,
            pltpu.SemaphoreType.DMA,
        ],
        compiler_params=pltpu.CompilerParams(needs_layout_passes=False),
    )(xt, indices)


def kernel(x, indices):
    xt = jnp.transpose(x, (0, 2, 3, 1)).reshape(N * C)
    yt = _shuffle(xt, indices)
    y = jnp.transpose(yt.reshape(NB, 32, 32, C), (0, 3, 1, 2))
    return (y, jnp.zeros((), dtype=x.dtype))


# speedup vs baseline: 8.8477x; 3.2715x over previous
import jax
import jax.numpy as jnp
from jax import lax
from jax.experimental import pallas as pl
from jax.experimental.pallas import tpu as pltpu
from jax.experimental.pallas import tpu_sc as plsc

NB = 16
C = 768
HW = 32 * 32
N = NB * HW
NC = 2
NS = 16
NW = NC * NS
ROWS_PER_W = N // NW
PCHUNK = 32
NCHUNK = ROWS_PER_W // PCHUNK
CG = C // 16


def _shuffle_body(x_hbm, idx_hbm, out_hbm, idx_v, in0, in1, out0, out1,
                  gsem0, gsem1, ssem0, ssem1):
    wid = lax.axis_index("s") * NC + lax.axis_index("c")
    base = wid * ROWS_PER_W

    pltpu.sync_copy(idx_hbm, idx_v)

    ins = (in0, in1)
    outs = (out0, out1)
    gsems = (gsem0, gsem1)
    ssems = (ssem0, ssem1)

    def gather(k):
        s = k % 2
        return pltpu.async_copy(
            x_hbm.at[pl.ds(base + k * PCHUNK, PCHUNK)], ins[s], gsems[s])

    def scatter(k):
        s = k % 2
        return pltpu.async_copy(
            outs[s], out_hbm.at[pl.ds(base + k * PCHUNK, PCHUNK)], ssems[s])

    def permute_chunk(in_buf, out_buf):
        @plsc.parallel_loop(0, CG)
        def body(j):
            csl = pl.ds(j * 16, 16)
            idxv = idx_v[csl]
            for p in range(PCHUNK):
                rowv = jnp.full((16,), p, dtype=jnp.int32)
                out_buf[p, csl] = plsc.load_gather(in_buf, [rowv, idxv])

    gathers = [None, None]
    scatters = [None, None]
    gathers[0] = gather(0)
    for k in range(NCHUNK):
        s = k % 2
        gathers[s].wait()
        if k + 1 < NCHUNK:
            gathers[1 - s] = gather(k + 1)
        if scatters[s] is not None:
            scatters[s].wait()
        permute_chunk(ins[s], outs[s])
        scatters[s] = scatter(k)
    scatters[0].wait()
    scatters[1].wait()


@jax.jit
def _shuffle(xt, indices):
    mesh = plsc.VectorSubcoreMesh(core_axis_name="c", subcore_axis_name="s")
    return pl.kernel(
        _shuffle_body,
        out_type=jax.ShapeDtypeStruct((N, C), jnp.float32),
        mesh=mesh,
        scratch_types=[
            pltpu.VMEM((C,), jnp.int32),
            pltpu.VMEM((PCHUNK, C), jnp.float32),
            pltpu.VMEM((PCHUNK, C), jnp.float32),
            pltpu.VMEM((PCHUNK, C), jnp.float32),
            pltpu.VMEM((PCHUNK, C), jnp.float32),
            pltpu.SemaphoreType.DMA,
            pltpu.SemaphoreType.DMA,
            pltpu.SemaphoreType.DMA,
            pltpu.SemaphoreType.DMA,
        ],
        compiler_params=pltpu.CompilerParams(needs_layout_passes=False),
    )(xt, indices)


def kernel(x, indices):
    xt = jnp.transpose(x, (0, 2, 3, 1)).reshape(N, C)
    yt = _shuffle(xt, indices)
    y = jnp.transpose(yt.reshape(NB, 32, 32, C), (0, 3, 1, 2))
    return (y, jnp.zeros((), dtype=x.dtype))
